# baseline (speedup 1.0000x reference)
import functools

import jax
import jax.numpy as jnp
from jax import lax
from jax.experimental import pallas as pl
from jax.experimental.pallas import tpu as pltpu
from jax.experimental.pallas import tpu_sc as plsc

NUM_TOKENS = 16384
HIDDEN = 768
EPS = 1e-7
LANES = 16
NUM_WORKERS = 32
TOK_PER_W = NUM_TOKENS // NUM_WORKERS
CHUNK = 64
NCHUNK = TOK_PER_W // CHUNK
DVECS = HIDDEN // LANES


def _rsqrt(x):
    i = lax.bitcast_convert_type(x, jnp.int32)
    i = jnp.full((LANES,), 0x5F3759DF, jnp.int32) - (i >> 1)
    y = lax.bitcast_convert_type(i, jnp.float32)
    half = x * 0.5
    for _ in range(3):
        y = y * (1.5 - half * y * y)
    return y


_GATHER_DNUMS = lax.GatherDimensionNumbers(
    offset_dims=(), collapsed_slice_dims=(0,), start_index_map=(0,))


def _lane_shuffle(v, idx):
    return lax.gather(v, idx[:, None], _GATHER_DNUMS, (1,),
                      mode=lax.GatherScatterMode.PROMISE_IN_BOUNDS)


def _allreduce_sum(v):
    lane = lax.iota(jnp.int32, LANES)
    for shift in (8, 4, 2, 1):
        v = v + _lane_shuffle(v, lane ^ shift)
    return v


def _sc_body(ids_hbm, pids_hbm, word_hbm, pos_hbm, gamma_hbm, beta_hbm,
             out_hbm, idx_w, idx_p, rows_w, rows_p, gam_v, bet_v,
             sem_w, sem_p):
    wid = lax.axis_index("s") * 2 + lax.axis_index("c")
    base = wid * TOK_PER_W

    pltpu.sync_copy(gamma_hbm, gam_v)
    pltpu.sync_copy(beta_hbm, bet_v)

    def chunk_body(g, _):
        cbase = base + g * CHUNK
        pltpu.sync_copy(ids_hbm.at[pl.ds(cbase, CHUNK)], idx_w)
        pltpu.sync_copy(pids_hbm.at[pl.ds(cbase, CHUNK)], idx_p)
        cw = pltpu.async_copy(word_hbm.at[idx_w], rows_w, sem_w)
        cp = pltpu.async_copy(pos_hbm.at[idx_p], rows_p, sem_p)
        cw.wait()
        cp.wait()

        def row_body(r, _):
            def acc_body(j, carry):
                acc, acc2 = carry
                v = (rows_w[r, pl.ds(j * LANES, LANES)]
                     + rows_p[r, pl.ds(j * LANES, LANES)])
                rows_w[r, pl.ds(j * LANES, LANES)] = v
                return acc + v, acc2 + v * v

            zero = jnp.zeros((LANES,), jnp.float32)
            acc, acc2 = lax.fori_loop(0, DVECS, acc_body, (zero, zero))
            mean = _allreduce_sum(acc) * (1.0 / HIDDEN)
            var = _allreduce_sum(acc2) * (1.0 / HIDDEN) - mean * mean
            rstd = _rsqrt(var + EPS)
            mrs = mean * rstd

            def norm_body(j, _):
                v = rows_w[r, pl.ds(j * LANES, LANES)]
                gmm = gam_v[pl.ds(j * LANES, LANES)]
                bta = bet_v[pl.ds(j * LANES, LANES)]
                rows_w[r, pl.ds(j * LANES, LANES)] = (
                    (v * rstd - mrs) * gmm + bta)
                return 0

            lax.fori_loop(0, DVECS, norm_body, 0)
            return 0

        lax.fori_loop(0, CHUNK, row_body, 0)
        pltpu.sync_copy(rows_w, out_hbm.at[pl.ds(cbase, CHUNK)])
        return 0

    lax.fori_loop(0, NCHUNK, chunk_body, 0)


def kernel(input_ids, seq_lens, position_ids, word_embeddings,
           position_embeddings, ln_gamma, ln_beta):
    del seq_lens
    mesh = plsc.VectorSubcoreMesh(core_axis_name="c", subcore_axis_name="s")
    k = functools.partial(
        pl.kernel,
        mesh=mesh,
        out_type=jax.ShapeDtypeStruct((NUM_TOKENS, HIDDEN), jnp.float32),
        scratch_types=[
            pltpu.VMEM((CHUNK,), jnp.int32),
            pltpu.VMEM((CHUNK,), jnp.int32),
            pltpu.VMEM((CHUNK, HIDDEN), jnp.float32),
            pltpu.VMEM((CHUNK, HIDDEN), jnp.float32),
            pltpu.VMEM((HIDDEN,), jnp.float32),
            pltpu.VMEM((HIDDEN,), jnp.float32),
            pltpu.SemaphoreType.DMA,
            pltpu.SemaphoreType.DMA,
        ],
    )(_sc_body)
    return k(input_ids.astype(jnp.int32), position_ids.astype(jnp.int32),
             word_embeddings, position_embeddings, ln_gamma, ln_beta)
